# baseline (speedup 1.0000x reference)
import functools

import jax
import jax.numpy as jnp
from jax import lax
from jax.experimental import pallas as pl
from jax.experimental.pallas import tpu as pltpu

N = 10000
T = 2
E = 320000
D = 128
K = 5
L = 2
OUT = 64


def _cheb_coef(row, col, w):
    w = jnp.where(row == col, 0.0, w)
    deg = jnp.zeros((N,), w.dtype).at[row].add(w)
    safe = jnp.where(deg > 0, deg, 1.0)
    dinv = jnp.where(deg > 0, 1.0 / jnp.sqrt(safe), 0.0)
    return -(dinv[row] * w * dinv[col])


def _spmm(coef, row, col, x):
    return jnp.zeros((N, x.shape[1]), x.dtype).at[col].add(coef[:, None] * x[row])


def _chebconv(x, W, b, row, col, coef):
    Tx0 = x
    out = Tx0 @ W[0]
    Tx1 = _spmm(coef, row, col, Tx0)
    out = out + Tx1 @ W[1]
    for k in range(2, K):
        Tx2 = 2.0 * _spmm(coef, row, col, Tx1) - Tx0
        out = out + Tx2 @ W[k]
        Tx0, Tx1 = Tx1, Tx2
    return out + b


def _gconvlstm(X, H, C, Wl, bl, wcl, gbl, row, col, coef):
    conv = lambda z, i: _chebconv(z, Wl[i], bl[i], row, col, coef)
    I = jax.nn.sigmoid(conv(X, 0) + conv(H, 1) + wcl[0] * C + gbl[0])
    Fg = jax.nn.sigmoid(conv(X, 2) + conv(H, 3) + wcl[1] * C + gbl[1])
    Tc = jnp.tanh(conv(X, 4) + conv(H, 5) + gbl[2])
    Cn = Fg * C + I * Tc
    O = jax.nn.sigmoid(conv(X, 6) + conv(H, 7) + wcl[2] * Cn + gbl[3])
    Hn = O * jnp.tanh(Cn)
    return Hn, Cn


def _decode_body(h_ref, w_ref, b_ref, o_ref):
    logits = jnp.dot(h_ref[...], w_ref[...],
                     preferred_element_type=jnp.float32) + b_ref[...]
    m = jnp.max(logits, axis=1, keepdims=True)
    z = logits - m
    lse = jnp.log(jnp.sum(jnp.exp(z), axis=1, keepdims=True))
    o_ref[...] = z - lse


@functools.partial(jax.jit, static_argnums=())
def _decode(h, dec_W, dec_b):
    BN = 400
    return pl.pallas_call(
        _decode_body,
        grid=(N // BN,),
        in_specs=[
            pl.BlockSpec((BN, D), lambda i: (i, 0)),
            pl.BlockSpec((D, OUT), lambda i: (0, 0)),
            pl.BlockSpec((1, OUT), lambda i: (0, 0)),
        ],
        out_specs=pl.BlockSpec((BN, OUT), lambda i: (i, 0)),
        out_shape=jax.ShapeDtypeStruct((N, OUT), jnp.float32),
    )(h, dec_W, dec_b.reshape(1, OUT))


def kernel(edge_index, edge_weight, emb, conv_w, conv_b, wc, gb, dec_W, dec_b):
    H = [jnp.ones((N, D), jnp.float32) for _ in range(L)]
    C = [jnp.ones((N, D), jnp.float32) for _ in range(L)]
    x = emb
    for t in range(T):
        row = edge_index[t, 0]
        col = edge_index[t, 1]
        coef = _cheb_coef(row, col, edge_weight[t])
        cur = x
        for l in range(L):
            h, c = _gconvlstm(cur, H[l], C[l], conv_w[l], conv_b[l],
                              wc[l], gb[l], row, col, coef)
            h = jax.nn.relu(h)
            H[l] = h
            C[l] = c
            cur = h
    return _decode(H[-1], dec_W, dec_b)


# speedup vs baseline: 1.9159x; 1.9159x over previous
import functools

import jax
import jax.numpy as jnp
from jax import lax
from jax.experimental import pallas as pl
from jax.experimental.pallas import tpu as pltpu
from jax.experimental.pallas import tpu_sc as plsc

N = 10000
T = 2
E = 320000
D = 128
K = 5
L = 2
OUT = 64

NC = 2
NS = 16
LANES = 16
NW = NC * NS
EPW = E // NW
CH = 80
NCHUNK = EPW // CH
NP = 10240
RPW = NP // NS

_sc_mesh = plsc.VectorSubcoreMesh(core_axis_name="c", subcore_axis_name="s")


def _bcast_lane(v, e):
    idx = jnp.full((LANES,), e, jnp.int32)
    return lax.gather(
        v, idx[:, None],
        dimension_numbers=lax.GatherDimensionNumbers(
            offset_dims=(), collapsed_slice_dims=(0,), start_index_map=(0,)),
        slice_sizes=(1,),
        mode=lax.GatherScatterMode.PROMISE_IN_BOUNDS)


def _sc_spmm_body(x_hbm, row_hbm, col_hbm, cf_hbm, out_hbm,
                  acc, rows, ridx, cidx, cfb, gsem, ssem):
    c = lax.axis_index("c")
    s = lax.axis_index("s")
    wid = s * NC + c

    zero = jnp.zeros((LANES,), jnp.float32)

    def zrow(i, _):
        for j in range(D // LANES):
            rows[i, pl.ds(j * LANES, LANES)] = zero
        return 0
    lax.fori_loop(0, CH, zrow, 0)
    rbase = s * RPW
    for k in range(RPW // CH):
        pltpu.sync_copy(rows, acc.at[pl.ds(rbase + k * CH, CH)])
    plsc.subcore_barrier()

    ebase = wid * EPW

    def chunk(i, _):
        base = ebase + i * CH
        pltpu.sync_copy(row_hbm.at[pl.ds(base, CH)], ridx)
        pltpu.sync_copy(col_hbm.at[pl.ds(base, CH)], cidx)
        pltpu.sync_copy(cf_hbm.at[pl.ds(base, CH)], cfb)
        pltpu.async_copy(x_hbm.at[ridx], rows, gsem).wait()

        def scale(g, _):
            cfv = cfb[pl.ds(g * LANES, LANES)]
            for e in range(LANES):
                b = _bcast_lane(cfv, e)
                r = g * LANES + e
                for j in range(D // LANES):
                    sl = pl.ds(j * LANES, LANES)
                    rows[r, sl] = rows[r, sl] * b
            return 0
        lax.fori_loop(0, CH // LANES, scale, 0)
        pltpu.async_copy(rows, acc.at[cidx], ssem, add=True).wait()
        return 0
    lax.fori_loop(0, NCHUNK, chunk, 0)

    plsc.subcore_barrier()
    RB = 64
    for k in range(RPW // RB):
        ds_ = pl.ds(rbase + k * RB, RB)
        pltpu.sync_copy(acc.at[ds_], rows.at[pl.ds(0, RB)])
        pltpu.sync_copy(rows.at[pl.ds(0, RB)], out_hbm.at[c, ds_])


_sc_spmm = functools.partial(
    pl.kernel,
    out_type=jax.ShapeDtypeStruct((NC, NP, D), jnp.float32),
    mesh=_sc_mesh,
    scratch_types=[
        pltpu.VMEM_SHARED((NP, D), jnp.float32),
        pltpu.VMEM((CH, D), jnp.float32),
        pltpu.VMEM((CH,), jnp.int32),
        pltpu.VMEM((CH,), jnp.int32),
        pltpu.VMEM((CH,), jnp.float32),
        pltpu.SemaphoreType.DMA,
        pltpu.SemaphoreType.DMA,
    ],
)(_sc_spmm_body)


def _spmm_sc(x, row, col, coef):
    parts = _sc_spmm(x, row, col, coef)
    return parts[0, :N] + parts[1, :N]


def _cheb_coef(row, col, w):
    w = jnp.where(row == col, 0.0, w)
    deg = jnp.zeros((N,), w.dtype).at[row].add(w)
    safe = jnp.where(deg > 0, deg, 1.0)
    dinv = jnp.where(deg > 0, 1.0 / jnp.sqrt(safe), 0.0)
    return -(dinv[row] * w * dinv[col])


def _spmm(coef, row, col, x):
    return _spmm_sc(x, row, col, coef)


def _chebconv(x, W, b, row, col, coef):
    Tx0 = x
    out = Tx0 @ W[0]
    Tx1 = _spmm(coef, row, col, Tx0)
    out = out + Tx1 @ W[1]
    for k in range(2, K):
        Tx2 = 2.0 * _spmm(coef, row, col, Tx1) - Tx0
        out = out + Tx2 @ W[k]
        Tx0, Tx1 = Tx1, Tx2
    return out + b


def _gconvlstm(X, H, C, Wl, bl, wcl, gbl, row, col, coef):
    conv = lambda z, i: _chebconv(z, Wl[i], bl[i], row, col, coef)
    I = jax.nn.sigmoid(conv(X, 0) + conv(H, 1) + wcl[0] * C + gbl[0])
    Fg = jax.nn.sigmoid(conv(X, 2) + conv(H, 3) + wcl[1] * C + gbl[1])
    Tc = jnp.tanh(conv(X, 4) + conv(H, 5) + gbl[2])
    Cn = Fg * C + I * Tc
    O = jax.nn.sigmoid(conv(X, 6) + conv(H, 7) + wcl[2] * Cn + gbl[3])
    Hn = O * jnp.tanh(Cn)
    return Hn, Cn


def _decode_body(h_ref, w_ref, b_ref, o_ref):
    logits = jnp.dot(h_ref[...], w_ref[...],
                     preferred_element_type=jnp.float32) + b_ref[...]
    m = jnp.max(logits, axis=1, keepdims=True)
    z = logits - m
    lse = jnp.log(jnp.sum(jnp.exp(z), axis=1, keepdims=True))
    o_ref[...] = z - lse


@functools.partial(jax.jit, static_argnums=())
def _decode(h, dec_W, dec_b):
    BN = 400
    return pl.pallas_call(
        _decode_body,
        grid=(N // BN,),
        in_specs=[
            pl.BlockSpec((BN, D), lambda i: (i, 0)),
            pl.BlockSpec((D, OUT), lambda i: (0, 0)),
            pl.BlockSpec((1, OUT), lambda i: (0, 0)),
        ],
        out_specs=pl.BlockSpec((BN, OUT), lambda i: (i, 0)),
        out_shape=jax.ShapeDtypeStruct((N, OUT), jnp.float32),
    )(h, dec_W, dec_b.reshape(1, OUT))


def kernel(edge_index, edge_weight, emb, conv_w, conv_b, wc, gb, dec_W, dec_b):
    H = [jnp.ones((N, D), jnp.float32) for _ in range(L)]
    C = [jnp.ones((N, D), jnp.float32) for _ in range(L)]
    x = emb
    for t in range(T):
        row = edge_index[t, 0]
        col = edge_index[t, 1]
        coef = _cheb_coef(row, col, edge_weight[t])
        cur = x
        for l in range(L):
            h, c = _gconvlstm(cur, H[l], C[l], conv_w[l], conv_b[l],
                              wc[l], gb[l], row, col, coef)
            h = jax.nn.relu(h)
            H[l] = h
            C[l] = c
            cur = h
    return _decode(H[-1], dec_W, dec_b)


# speedup vs baseline: 2.6839x; 1.4009x over previous
import functools

import jax
import jax.numpy as jnp
from jax import lax
from jax.experimental import pallas as pl
from jax.experimental.pallas import tpu as pltpu
from jax.experimental.pallas import tpu_sc as plsc

N = 10000
T = 2
E = 320000
D = 128
K = 5
L = 2
OUT = 64

NC = 2
NS = 16
LANES = 16
NW = NC * NS
EPW = E // NW
CH = 64
NFULL = EPW // CH
TAIL = EPW - NFULL * CH
EPAD = NFULL * CH + CH
NP = 10240
RPW = NP // NS
RSH = 14
CMASK = (1 << RSH) - 1

_sc_mesh = plsc.VectorSubcoreMesh(core_axis_name="c", subcore_axis_name="s")


def _bcast_lane(v, e):
    idx = jnp.full((LANES,), e, jnp.int32)
    return lax.gather(
        v, idx[:, None],
        dimension_numbers=lax.GatherDimensionNumbers(
            offset_dims=(), collapsed_slice_dims=(0,), start_index_map=(0,)),
        slice_sizes=(1,),
        mode=lax.GatherScatterMode.PROMISE_IN_BOUNDS)


def _sc_spmm_body(x_hbm, pk_hbm, cf_hbm, out_hbm, acc, pkb, cfb,
                  rows0, rows1, rows2, ri0, ri1, ri2, ci0, ci1, ci2,
                  gs0, gs1, gs2, ss0, ss1, ss2):
    c = lax.axis_index("c")
    s = lax.axis_index("s")
    wid = s * NC + c
    rowsb = (rows0, rows1, rows2)
    rib = (ri0, ri1, ri2)
    cib = (ci0, ci1, ci2)
    gsem = (gs0, gs1, gs2)
    ssem = (ss0, ss1, ss2)
    zero = jnp.zeros((LANES,), jnp.float32)

    def zrow(i, _):
        for j in range(D // LANES):
            rows0[i, pl.ds(j * LANES, LANES)] = zero
        return 0
    lax.fori_loop(0, CH, zrow, 0)
    rbase = s * RPW
    for k in range(RPW // CH):
        pltpu.sync_copy(rows0, acc.at[pl.ds(rbase + k * CH, CH)])

    ebase = wid * EPW
    pltpu.sync_copy(pk_hbm.at[pl.ds(ebase, EPW)], pkb.at[pl.ds(0, EPW)])
    pltpu.sync_copy(cf_hbm.at[pl.ds(ebase, EPW)], cfb.at[pl.ds(0, EPW)])
    for g in range((EPAD - EPW) // LANES):
        sl = pl.ds(EPW + g * LANES, LANES)
        pkb[sl] = jnp.zeros((LANES,), jnp.int32)
        cfb[sl] = zero

    def unpack(kp1, b):
        for g in range(CH // LANES):
            pk = pkb[pl.ds(kp1 * CH + g * LANES, LANES)]
            sl = pl.ds(g * LANES, LANES)
            rib[b][sl] = lax.shift_right_logical(pk, RSH)
            cib[b][sl] = lax.bitwise_and(pk, CMASK)

    def scale(k, b):
        def gbody(g, _):
            cfv = cfb[pl.ds(k * CH + g * LANES, LANES)]
            for e in range(LANES):
                bc = _bcast_lane(cfv, e)
                r = g * LANES + e
                for j in range(D // LANES):
                    sl = pl.ds(j * LANES, LANES)
                    rowsb[b][r, sl] = rowsb[b][r, sl] * bc
            return 0
        lax.fori_loop(0, CH // LANES, gbody, 0)

    unpack(0, 0)
    pltpu.async_copy(x_hbm.at[ri0], rows0, gs0)
    plsc.subcore_barrier()

    def body(i, _):
        for j in range(3):
            k = 3 * i + j
            bn = (j + 1) % 3
            if j == 2:
                pltpu.make_async_copy(
                    rowsb[bn], acc.at[pl.ds(0, CH)], ssem[bn]).wait()
            else:
                @pl.when(i > 0)
                def _():
                    pltpu.make_async_copy(
                        rowsb[bn], acc.at[pl.ds(0, CH)], ssem[bn]).wait()
            unpack(k + 1, bn)
            pltpu.async_copy(x_hbm.at[rib[bn]], rowsb[bn], gsem[bn])
            pltpu.make_async_copy(
                x_hbm.at[pl.ds(0, CH)], rowsb[j], gsem[j]).wait()
            scale(k, j)
            pltpu.async_copy(rowsb[j], acc.at[cib[j]], ssem[j], add=True)
        return 0
    lax.fori_loop(0, NFULL // 3, body, 0)

    pltpu.make_async_copy(x_hbm.at[pl.ds(0, CH)], rows0, gs0).wait()
    scale(NFULL, 0)
    pltpu.async_copy(rows0, acc.at[ci0], ss0, add=True)
    for b in range(3):
        pltpu.make_async_copy(rowsb[b], acc.at[pl.ds(0, CH)], ssem[b]).wait()

    plsc.subcore_barrier()
    for k in range(RPW // CH):
        ds_ = pl.ds(rbase + k * CH, CH)
        pltpu.sync_copy(acc.at[ds_], rows0)
        pltpu.sync_copy(rows0, out_hbm.at[c, ds_])


_sc_spmm = functools.partial(
    pl.kernel,
    out_type=jax.ShapeDtypeStruct((NC, NP, D), jnp.float32),
    mesh=_sc_mesh,
    scratch_types=(
        [pltpu.VMEM_SHARED((NP, D), jnp.float32),
         pltpu.VMEM((EPAD,), jnp.int32),
         pltpu.VMEM((EPAD,), jnp.float32)]
        + [pltpu.VMEM((CH, D), jnp.float32)] * 3
        + [pltpu.VMEM((CH,), jnp.int32)] * 6
        + [pltpu.SemaphoreType.DMA] * 6
    ),
)(_sc_spmm_body)


def _spmm_sc(x, packed, coef):
    parts = _sc_spmm(x, packed, coef)
    return parts[0, :N] + parts[1, :N]


def _cheb_coef(row, col, w):
    w = jnp.where(row == col, 0.0, w)
    deg = jnp.zeros((N,), w.dtype).at[row].add(w)
    safe = jnp.where(deg > 0, deg, 1.0)
    dinv = jnp.where(deg > 0, 1.0 / jnp.sqrt(safe), 0.0)
    return -(dinv[row] * w * dinv[col])


def _chebconv(x, W, b, pk, coef):
    Tx0 = x
    out = Tx0 @ W[0]
    Tx1 = _spmm_sc(Tx0, pk, coef)
    out = out + Tx1 @ W[1]
    for k in range(2, K):
        Tx2 = 2.0 * _spmm_sc(Tx1, pk, coef) - Tx0
        out = out + Tx2 @ W[k]
        Tx0, Tx1 = Tx1, Tx2
    return out + b


def _gconvlstm(X, H, C, Wl, bl, wcl, gbl, pk, coef):
    conv = lambda z, i: _chebconv(z, Wl[i], bl[i], pk, coef)
    I = jax.nn.sigmoid(conv(X, 0) + conv(H, 1) + wcl[0] * C + gbl[0])
    Fg = jax.nn.sigmoid(conv(X, 2) + conv(H, 3) + wcl[1] * C + gbl[1])
    Tc = jnp.tanh(conv(X, 4) + conv(H, 5) + gbl[2])
    Cn = Fg * C + I * Tc
    O = jax.nn.sigmoid(conv(X, 6) + conv(H, 7) + wcl[2] * Cn + gbl[3])
    Hn = O * jnp.tanh(Cn)
    return Hn, Cn


def _decode_body(h_ref, w_ref, b_ref, o_ref):
    logits = jnp.dot(h_ref[...], w_ref[...],
                     preferred_element_type=jnp.float32) + b_ref[...]
    m = jnp.max(logits, axis=1, keepdims=True)
    z = logits - m
    lse = jnp.log(jnp.sum(jnp.exp(z), axis=1, keepdims=True))
    o_ref[...] = z - lse


@functools.partial(jax.jit, static_argnums=())
def _decode(h, dec_W, dec_b):
    BN = 400
    return pl.pallas_call(
        _decode_body,
        grid=(N // BN,),
        in_specs=[
            pl.BlockSpec((BN, D), lambda i: (i, 0)),
            pl.BlockSpec((D, OUT), lambda i: (0, 0)),
            pl.BlockSpec((1, OUT), lambda i: (0, 0)),
        ],
        out_specs=pl.BlockSpec((BN, OUT), lambda i: (i, 0)),
        out_shape=jax.ShapeDtypeStruct((N, OUT), jnp.float32),
    )(h, dec_W, dec_b.reshape(1, OUT))


def kernel(edge_index, edge_weight, emb, conv_w, conv_b, wc, gb, dec_W, dec_b):
    H = [jnp.ones((N, D), jnp.float32) for _ in range(L)]
    C = [jnp.ones((N, D), jnp.float32) for _ in range(L)]
    x = emb
    for t in range(T):
        row = edge_index[t, 0]
        col = edge_index[t, 1]
        coef = _cheb_coef(row, col, edge_weight[t])
        pk = jnp.left_shift(row, RSH) | col
        cur = x
        for l in range(L):
            h, c = _gconvlstm(cur, H[l], C[l], conv_w[l], conv_b[l],
                              wc[l], gb[l], pk, coef)
            h = jax.nn.relu(h)
            H[l] = h
            C[l] = c
            cur = h
    return _decode(H[-1], dec_W, dec_b)


# speedup vs baseline: 7.0616x; 2.6311x over previous
import functools

import jax
import jax.numpy as jnp
from jax import lax
from jax.experimental import pallas as pl
from jax.experimental.pallas import tpu as pltpu
from jax.experimental.pallas import tpu_sc as plsc

N = 10000
T = 2
E = 320000
D = 128
K = 5
L = 2
OUT = 64

NC = 2
NS = 16
LANES = 16
NW = NC * NS
EPW = E // NW
CH = 64
NFULL = EPW // CH
EPAD = NFULL * CH + CH
NP = 10240
RPW = NP // NS
RSH = 14
CMASK = (1 << RSH) - 1
EC = 2000
NEC = EPW // EC

_sc_mesh = plsc.VectorSubcoreMesh(core_axis_name="c", subcore_axis_name="s")


def _bcast_lane(v, e):
    idx = jnp.full((LANES,), e, jnp.int32)
    return lax.gather(
        v, idx[:, None],
        dimension_numbers=lax.GatherDimensionNumbers(
            offset_dims=(), collapsed_slice_dims=(0,), start_index_map=(0,)),
        slice_sizes=(1,),
        mode=lax.GatherScatterMode.PROMISE_IN_BOUNDS)


def _sc_spmm_body(x_hbm, pk_hbm, cf_hbm, out_hbm, acc, pkb, cfb,
                  rows0, rows1, rows2, ri0, ri1, ri2, ci0, ci1, ci2,
                  gs0, gs1, gs2, ss0, ss1, ss2, zsem):
    c = lax.axis_index("c")
    s = lax.axis_index("s")
    wid = s * NC + c
    rowsb = (rows0, rows1, rows2)
    rib = (ri0, ri1, ri2)
    cib = (ci0, ci1, ci2)
    gsem = (gs0, gs1, gs2)
    ssem = (ss0, ss1, ss2)
    zero = jnp.zeros((LANES,), jnp.float32)

    def zrow(i, _):
        for j in range(D // LANES):
            rows0[i, pl.ds(j * LANES, LANES)] = zero
        return 0
    lax.fori_loop(0, CH, zrow, 0)
    rbase = s * RPW
    for k in range(RPW // CH):
        pltpu.async_copy(rows0, acc.at[pl.ds(rbase + k * CH, CH)], zsem)

    ebase = wid * EPW
    pltpu.sync_copy(pk_hbm.at[pl.ds(ebase, EPW)], pkb.at[pl.ds(0, EPW)])
    pltpu.sync_copy(cf_hbm.at[pl.ds(ebase, EPW)], cfb.at[pl.ds(0, EPW)])
    for g in range((EPAD - EPW) // LANES):
        sl = pl.ds(EPW + g * LANES, LANES)
        pkb[sl] = jnp.zeros((LANES,), jnp.int32)
        cfb[sl] = zero

    def unpack(kp1, b):
        for g in range(CH // LANES):
            pk = pkb[pl.ds(kp1 * CH + g * LANES, LANES)]
            sl = pl.ds(g * LANES, LANES)
            rib[b][sl] = lax.shift_right_logical(pk, RSH)
            cib[b][sl] = lax.bitwise_and(pk, CMASK)

    def scale(k, b):
        def gbody(g, _):
            cfv = cfb[pl.ds(k * CH + g * LANES, LANES)]
            for e in range(LANES):
                bc = _bcast_lane(cfv, e)
                r = g * LANES + e
                for j in range(D // LANES):
                    sl = pl.ds(j * LANES, LANES)
                    rowsb[b][r, sl] = rowsb[b][r, sl] * bc
            return 0
        lax.fori_loop(0, CH // LANES, gbody, 0)

    unpack(0, 0)
    for k in range(RPW // CH):
        pltpu.make_async_copy(rows1, acc.at[pl.ds(0, CH)], zsem).wait()
    pltpu.async_copy(x_hbm.at[ri0], rows0, gs0)
    plsc.subcore_barrier()

    def body(i, _):
        for j in range(3):
            k = 3 * i + j
            bn = (j + 1) % 3
            if j == 2:
                pltpu.make_async_copy(
                    rowsb[bn], acc.at[pl.ds(0, CH)], ssem[bn]).wait()
            else:
                @pl.when(i > 0)
                def _():
                    pltpu.make_async_copy(
                        rowsb[bn], acc.at[pl.ds(0, CH)], ssem[bn]).wait()
            unpack(k + 1, bn)
            pltpu.async_copy(x_hbm.at[rib[bn]], rowsb[bn], gsem[bn])
            pltpu.make_async_copy(
                x_hbm.at[pl.ds(0, CH)], rowsb[j], gsem[j]).wait()
            scale(k, j)
            pltpu.async_copy(rowsb[j], acc.at[cib[j]], ssem[j], add=True)
        return 0
    lax.fori_loop(0, NFULL // 3, body, 0)

    pltpu.make_async_copy(x_hbm.at[pl.ds(0, CH)], rows0, gs0).wait()
    scale(NFULL, 0)
    pltpu.async_copy(rows0, acc.at[ci0], ss0, add=True)
    for b in range(3):
        pltpu.make_async_copy(rowsb[b], acc.at[pl.ds(0, CH)], ssem[b]).wait()

    plsc.subcore_barrier()
    for k in range(RPW // CH):
        b = k % 3
        ds_ = pl.ds(rbase + k * CH, CH)
        if k >= 3:
            pltpu.make_async_copy(
                rowsb[b], out_hbm.at[c, pl.ds(0, CH)], ssem[b]).wait()
        pltpu.sync_copy(acc.at[ds_], rowsb[b])
        pltpu.async_copy(rowsb[b], out_hbm.at[c, ds_], ssem[b])
    for b in range(3):
        pltpu.make_async_copy(
            rowsb[b], out_hbm.at[c, pl.ds(0, CH)], ssem[b]).wait()


_sc_spmm = functools.partial(
    pl.kernel,
    out_type=jax.ShapeDtypeStruct((NC, NP, D), jnp.float32),
    mesh=_sc_mesh,
    scratch_types=(
        [pltpu.VMEM_SHARED((NP, D), jnp.float32),
         pltpu.VMEM((EPAD,), jnp.int32),
         pltpu.VMEM((EPAD,), jnp.float32)]
        + [pltpu.VMEM((CH, D), jnp.float32)] * 3
        + [pltpu.VMEM((CH,), jnp.int32)] * 6
        + [pltpu.SemaphoreType.DMA] * 7
    ),
)(_sc_spmm_body)


SUB = 80
NSUB = EPW // SUB


def _sc_deg_body(row_hbm, col_hbm, w_hbm, out_hbm, acc,
                 rowb, colb, wb, zbuf, rs0, rs1, wv0, wv1, sm0, sm1):
    c = lax.axis_index("c")
    s = lax.axis_index("s")
    wid = s * NC + c
    rsm = (rs0, rs1)
    wvm = (wv0, wv1)
    sem = (sm0, sm1)
    zero = jnp.zeros((LANES,), jnp.float32)

    for g in range(RPW // LANES):
        zbuf[pl.ds(g * LANES, LANES)] = zero
    pltpu.sync_copy(zbuf, acc.at[pl.ds(s * RPW, RPW)])

    ebase = wid * EPW
    pltpu.sync_copy(row_hbm.at[pl.ds(ebase, EPW)], rowb)
    pltpu.sync_copy(col_hbm.at[pl.ds(ebase, EPW)], colb)
    pltpu.sync_copy(w_hbm.at[pl.ds(ebase, EPW)], wb)
    plsc.subcore_barrier()

    def stage(i, b):
        for g in range(SUB // LANES):
            eb = pl.ds(i * SUB + g * LANES, LANES)
            sl = pl.ds(g * LANES, LANES)
            r16 = rowb[eb]
            rsm[b][sl] = r16
            wvm[b][sl] = jnp.where(r16 == colb[eb], 0.0, wb[eb])

    def body(i2, _):
        for b in range(2):
            i = 2 * i2 + b

            @pl.when(i2 > 0)
            def _():
                pltpu.make_async_copy(
                    wvm[b], acc.at[pl.ds(0, SUB)], sem[b]).wait()
            stage(i, b)
            pltpu.async_copy(wvm[b], acc.at[rsm[b]], sem[b], add=True)
        return 0
    lax.fori_loop(0, NSUB // 2, body, 0)
    pltpu.make_async_copy(wv0, acc.at[pl.ds(0, SUB)], sm0).wait()
    stage(NSUB - 1, 0)
    pltpu.async_copy(wv0, acc.at[rs0], sm0, add=True)
    pltpu.make_async_copy(wv0, acc.at[pl.ds(0, SUB)], sm0).wait()
    pltpu.make_async_copy(wv1, acc.at[pl.ds(0, SUB)], sm1).wait()

    plsc.subcore_barrier()
    pltpu.sync_copy(acc.at[pl.ds(s * RPW, RPW)], zbuf)
    pltpu.sync_copy(zbuf, out_hbm.at[c, pl.ds(s * RPW, RPW)])


_sc_deg = functools.partial(
    pl.kernel,
    out_type=jax.ShapeDtypeStruct((NC, NP), jnp.float32),
    mesh=_sc_mesh,
    scratch_types=(
        [pltpu.VMEM_SHARED((NP,), jnp.float32),
         pltpu.VMEM((EPW,), jnp.int32),
         pltpu.VMEM((EPW,), jnp.int32),
         pltpu.VMEM((EPW,), jnp.float32),
         pltpu.VMEM((RPW,), jnp.float32)]
        + [pltpu.VMEM((SUB,), jnp.int32)] * 2
        + [pltpu.VMEM((SUB,), jnp.float32)] * 2
        + [pltpu.SemaphoreType.DMA] * 2
    ),
)(_sc_deg_body)


def _sc_coef_body(row_hbm, col_hbm, w_hbm, dinv_hbm, cf_hbm, pk_hbm,
                  rowb, colb, wb, cfbig, pkbig,
                  ri0, ri1, ci0, ci1, dr0, dr1, dc0, dc1, sm0, sm1):
    c = lax.axis_index("c")
    s = lax.axis_index("s")
    wid = s * NC + c
    rim = (ri0, ri1)
    cim = (ci0, ci1)
    drm = (dr0, dr1)
    dcm = (dc0, dc1)
    sem = (sm0, sm1)

    ebase = wid * EPW
    pltpu.sync_copy(row_hbm.at[pl.ds(ebase, EPW)], rowb)
    pltpu.sync_copy(col_hbm.at[pl.ds(ebase, EPW)], colb)
    pltpu.sync_copy(w_hbm.at[pl.ds(ebase, EPW)], wb)

    def stage(i, b):
        for g in range(SUB // LANES):
            eb = pl.ds(i * SUB + g * LANES, LANES)
            sl = pl.ds(g * LANES, LANES)
            rim[b][sl] = rowb[eb]
            cim[b][sl] = colb[eb]

    def fire(b):
        pltpu.async_copy(dinv_hbm.at[rim[b]], drm[b], sem[b])
        pltpu.async_copy(dinv_hbm.at[cim[b]], dcm[b], sem[b])

    def drain(b):
        pltpu.make_async_copy(dinv_hbm.at[pl.ds(0, SUB)], drm[b],
                              sem[b]).wait()
        pltpu.make_async_copy(dinv_hbm.at[pl.ds(0, SUB)], dcm[b],
                              sem[b]).wait()

    def compute(i, b):
        for g in range(SUB // LANES):
            eb = pl.ds(i * SUB + g * LANES, LANES)
            sl = pl.ds(g * LANES, LANES)
            r16 = rowb[eb]
            c16 = colb[eb]
            wz = jnp.where(r16 == c16, 0.0, wb[eb])
            cfbig[eb] = -(drm[b][sl] * wz * dcm[b][sl])
            pkbig[eb] = lax.bitwise_or(lax.shift_left(r16, RSH), c16)

    stage(0, 0)
    fire(0)

    def body(i2, _):
        for b in range(2):
            i = 2 * i2 + b
            bn = 1 - b
            stage(i + 1, bn)
            fire(bn)
            drain(b)
            compute(i, b)
        return 0
    lax.fori_loop(0, (NSUB - 1) // 2, body, 0)
    drain(0)
    compute(NSUB - 1, 0)

    pltpu.sync_copy(cfbig, cf_hbm.at[pl.ds(ebase, EPW)])
    pltpu.sync_copy(pkbig, pk_hbm.at[pl.ds(ebase, EPW)])


_sc_coef = functools.partial(
    pl.kernel,
    out_type=(jax.ShapeDtypeStruct((E,), jnp.float32),
              jax.ShapeDtypeStruct((E,), jnp.int32)),
    mesh=_sc_mesh,
    scratch_types=(
        [pltpu.VMEM((EPW,), jnp.int32),
         pltpu.VMEM((EPW,), jnp.int32),
         pltpu.VMEM((EPW,), jnp.float32),
         pltpu.VMEM((EPW,), jnp.float32),
         pltpu.VMEM((EPW,), jnp.int32)]
        + [pltpu.VMEM((SUB,), jnp.int32)] * 4
        + [pltpu.VMEM((SUB,), jnp.float32)] * 4
        + [pltpu.SemaphoreType.DMA] * 2
    ),
)(_sc_coef_body)


def _dinv_body(d_ref, o_ref):
    deg = d_ref[0] + d_ref[1]
    safe = jnp.where(deg > 0, deg, 1.0)
    o_ref[...] = jnp.where(deg > 0, lax.rsqrt(safe), 0.0)


def _tc_dinv(degp):
    return pl.pallas_call(
        _dinv_body,
        out_shape=jax.ShapeDtypeStruct((NP // D, D), jnp.float32),
    )(degp)


BN = 1000


def _comb1_body(p0_ref, p1_ref, o_ref):
    o_ref[...] = p0_ref[...] + p1_ref[...]


def _comb2_body(p0_ref, p1_ref, t_ref, o_ref):
    o_ref[...] = 2.0 * (p0_ref[...] + p1_ref[...]) - t_ref[...]


def _tc_comb1(parts):
    return pl.pallas_call(
        _comb1_body,
        grid=(N // BN,),
        in_specs=[pl.BlockSpec((BN, D), lambda i: (i, 0))] * 2,
        out_specs=pl.BlockSpec((BN, D), lambda i: (i, 0)),
        out_shape=jax.ShapeDtypeStruct((N, D), jnp.float32),
    )(parts[0], parts[1])


def _tc_comb2(parts, tprev):
    return pl.pallas_call(
        _comb2_body,
        grid=(N // BN,),
        in_specs=[pl.BlockSpec((BN, D), lambda i: (i, 0))] * 3,
        out_specs=pl.BlockSpec((BN, D), lambda i: (i, 0)),
        out_shape=jax.ShapeDtypeStruct((N, D), jnp.float32),
    )(parts[0], parts[1], tprev)


def _gates_body(tx0, tx1, tx2, tx3, tx4, th0, th1, th2, th3, th4,
                c_ref, wx_ref, wh_ref, b_ref, wc_ref, h_ref, cn_ref):
    txs = (tx0, tx1, tx2, tx3, tx4)
    ths = (th0, th1, th2, th3, th4)
    z = b_ref[0:1, :]
    for k in range(K):
        wsl = pl.ds(k * D, D)
        z = z + jnp.dot(txs[k][...], wx_ref[wsl, :],
                        preferred_element_type=jnp.float32)
        z = z + jnp.dot(ths[k][...], wh_ref[wsl, :],
                        preferred_element_type=jnp.float32)
    Cp = c_ref[...]
    Ig = jax.nn.sigmoid(z[:, 0 * D:1 * D] + wc_ref[0:1, :] * Cp)
    Fg = jax.nn.sigmoid(z[:, 1 * D:2 * D] + wc_ref[1:2, :] * Cp)
    Tg = jnp.tanh(z[:, 2 * D:3 * D])
    Cn = Fg * Cp + Ig * Tg
    Og = jax.nn.sigmoid(z[:, 3 * D:4 * D] + wc_ref[2:3, :] * Cn)
    cn_ref[...] = Cn
    h_ref[...] = jnp.maximum(Og * jnp.tanh(Cn), 0.0)


def _tc_gates(txs, ths, C, Wx, Wh, bias, wcp):
    blk = pl.BlockSpec((BN, D), lambda i: (i, 0))
    outs = pl.pallas_call(
        _gates_body,
        grid=(N // BN,),
        in_specs=[blk] * 11 + [
            pl.BlockSpec((K * D, 4 * D), lambda i: (0, 0)),
            pl.BlockSpec((K * D, 4 * D), lambda i: (0, 0)),
            pl.BlockSpec((8, 4 * D), lambda i: (0, 0)),
            pl.BlockSpec((8, D), lambda i: (0, 0)),
        ],
        out_specs=[blk, blk],
        out_shape=[jax.ShapeDtypeStruct((N, D), jnp.float32)] * 2,
    )(*txs, *ths, C, Wx, Wh, bias, wcp)
    return outs[0], outs[1]


def _decode_body(h_ref, w_ref, b_ref, o_ref):
    logits = jnp.dot(h_ref[...], w_ref[...],
                     preferred_element_type=jnp.float32) + b_ref[0:1, :]
    m = jnp.max(logits, axis=1, keepdims=True)
    z = logits - m
    lse = jnp.log(jnp.sum(jnp.exp(z), axis=1, keepdims=True))
    o_ref[...] = z - lse


def _tc_decode(h, dec_W, dec_bp):
    return pl.pallas_call(
        _decode_body,
        grid=(N // BN,),
        in_specs=[
            pl.BlockSpec((BN, D), lambda i: (i, 0)),
            pl.BlockSpec((D, OUT), lambda i: (0, 0)),
            pl.BlockSpec((8, OUT), lambda i: (0, 0)),
        ],
        out_specs=pl.BlockSpec((BN, OUT), lambda i: (i, 0)),
        out_shape=jax.ShapeDtypeStruct((N, OUT), jnp.float32),
    )(h, dec_W, dec_bp)


def _basis(x, pk, cf):
    t0 = x
    t1 = _tc_comb1(_sc_spmm(t0, pk, cf))
    ts = [t0, t1]
    for _ in range(2, K):
        ts.append(_tc_comb2(_sc_spmm(ts[-1], pk, cf), ts[-2]))
    return ts


def kernel(edge_index, edge_weight, emb, conv_w, conv_b, wc, gb, dec_W, dec_b):
    Wx, Wh, bias, wcp = [], [], [], []
    for l in range(L):
        wxl = jnp.concatenate([conv_w[l, g] for g in (0, 2, 4, 6)], axis=-1)
        whl = jnp.concatenate([conv_w[l, g] for g in (1, 3, 5, 7)], axis=-1)
        Wx.append(wxl.reshape(K * D, 4 * D))
        Wh.append(whl.reshape(K * D, 4 * D))
        bsum = conv_b[l].reshape(4, 2, D).sum(1) + gb[l]
        bias.append(jnp.zeros((8, 4 * D), jnp.float32)
                    .at[0].set(bsum.reshape(4 * D)))
        wcp.append(jnp.zeros((8, D), jnp.float32).at[0:3].set(wc[l]))
    dec_bp = jnp.zeros((8, OUT), jnp.float32).at[0].set(dec_b)

    H = [jnp.ones((N, D), jnp.float32) for _ in range(L)]
    C = [jnp.ones((N, D), jnp.float32) for _ in range(L)]
    for t in range(T):
        row = edge_index[t, 0]
        col = edge_index[t, 1]
        degp = _sc_deg(row, col, edge_weight[t])
        dinv = _tc_dinv(degp.reshape(NC, NP // D, D)).reshape(NP)
        cf, pk = _sc_coef(row, col, edge_weight[t], dinv)
        cur = emb
        for l in range(L):
            txs = _basis(cur, pk, cf)
            ths = _basis(H[l], pk, cf)
            h, c = _tc_gates(txs, ths, C[l], Wx[l], Wh[l], bias[l], wcp[l])
            H[l] = h
            C[l] = c
            cur = h
    return _tc_decode(H[-1], dec_W, dec_bp)
